# baseline (speedup 1.0000x reference)
import functools

import jax
import jax.numpy as jnp
from jax import lax
from jax.experimental import pallas as pl
from jax.experimental.pallas import tpu as pltpu
from jax.experimental.pallas import tpu_sc as plsc

B = 16384
D = 64
H = 4 * D
NC, NS = 2, 16
NW = NC * NS
BPW = B // NW

_mesh = plsc.VectorSubcoreMesh(
    core_axis_name="c", subcore_axis_name="s", num_cores=NC, num_subcores=NS
)


@functools.partial(
    pl.kernel,
    out_type=tuple(jax.ShapeDtypeStruct((B, D), jnp.float32) for _ in range(4)),
    mesh=_mesh,
    scratch_types=[
        pltpu.VMEM((BPW,), jnp.int32),
        pltpu.VMEM((BPW, D), jnp.float32),
        pltpu.SemaphoreType.DMA,
    ],
    compiler_params=pltpu.CompilerParams(use_tc_tiling_on_sc=False),
)
def _sc_gather(tg, ta, tal, tc, ig, ia, ial, ic, o0, o1, o2, o3,
               idx_v, rows_v, sem):
    wid = lax.axis_index("s") * NC + lax.axis_index("c")
    base = wid * BPW
    for tbl, idx_hbm, out_hbm in ((tg, ig, o0), (ta, ia, o1),
                                  (tal, ial, o2), (tc, ic, o3)):
        pltpu.sync_copy(idx_hbm.at[pl.ds(base, BPW)], idx_v)
        pltpu.async_copy(tbl.at[idx_v], rows_v, sem).wait()
        pltpu.sync_copy(rows_v, out_hbm.at[pl.ds(base, BPW)])


TB = 2048


def _mm_body(e0, e1, e2, e3, w_ref, b_ref, o_ref):
    x = jnp.concatenate([e0[...], e1[...], e2[...], e3[...]], axis=1)
    acc = lax.dot_general(x, w_ref[...], (((1,), (1,)), ((), ())),
                          preferred_element_type=jnp.float32)
    o_ref[...] = acc + b_ref[...]


_mm = pl.pallas_call(
    _mm_body,
    grid=(B // TB,),
    in_specs=[
        pl.BlockSpec((TB, D), lambda i: (i, 0)),
        pl.BlockSpec((TB, D), lambda i: (i, 0)),
        pl.BlockSpec((TB, D), lambda i: (i, 0)),
        pl.BlockSpec((TB, D), lambda i: (i, 0)),
        pl.BlockSpec((H, H), lambda i: (0, 0)),
        pl.BlockSpec((1, H), lambda i: (0, 0)),
    ],
    out_specs=pl.BlockSpec((TB, H), lambda i: (i, 0)),
    out_shape=jax.ShapeDtypeStruct((B, H), jnp.float32),
    compiler_params=pltpu.CompilerParams(
        dimension_semantics=("parallel",),
    ),
)


def kernel(emb_genre, emb_artist, emb_album, emb_country, W, b,
           idx_genre, idx_artist, idx_album, idx_country):
    e0, e1, e2, e3 = _sc_gather(
        emb_genre, emb_artist, emb_album, emb_country,
        idx_genre.astype(jnp.int32), idx_artist.astype(jnp.int32),
        idx_album.astype(jnp.int32), idx_country.astype(jnp.int32),
    )
    return _mm(e0, e1, e2, e3, W, b.reshape(1, H))


# speedup vs baseline: 1.6966x; 1.6966x over previous
import functools

import jax
import jax.numpy as jnp
from jax import lax
from jax.experimental import pallas as pl
from jax.experimental.pallas import tpu as pltpu
from jax.experimental.pallas import tpu_sc as plsc

B = 16384
D = 64
H = 4 * D
NC, NS = 2, 16
NW = NC * NS
BPW = B // NW

_mesh = plsc.VectorSubcoreMesh(
    core_axis_name="c", subcore_axis_name="s", num_cores=NC, num_subcores=NS
)


@functools.partial(
    pl.kernel,
    out_type=(
        jax.ShapeDtypeStruct((B, 2 * D), jnp.float32),
        jax.ShapeDtypeStruct((B, 2 * D), jnp.float32),
    ),
    mesh=_mesh,
    scratch_types=[
        pltpu.VMEM((BPW,), jnp.int32),
        pltpu.VMEM((BPW,), jnp.int32),
        pltpu.VMEM((BPW, 2 * D), jnp.float32),
        pltpu.SemaphoreType.DMA,
    ],
)
def _sc_gather(tg, ta, tal, tc_, ig, ia, ial, ic, out01, out23,
               idx_l, idx_r, rows_v, sem):
    wid = lax.axis_index("s") * NC + lax.axis_index("c")
    base = wid * BPW
    for tbl_l, idx_hbm_l, tbl_r, idx_hbm_r, out_hbm in (
        (tg, ig, ta, ia, out01),
        (tal, ial, tc_, ic, out23),
    ):
        pltpu.sync_copy(idx_hbm_l.at[pl.ds(base, BPW)], idx_l)
        pltpu.sync_copy(idx_hbm_r.at[pl.ds(base, BPW)], idx_r)

        def body(g, _, tbl_l=tbl_l, tbl_r=tbl_r):
            i0 = g * 16
            vl = idx_l[pl.ds(i0, 16)]
            vr = idx_r[pl.ds(i0, 16)]
            for j in range(16):
                pltpu.async_copy(
                    tbl_l.at[vl[j]], rows_v.at[i0 + j, pl.ds(0, D)], sem)
                pltpu.async_copy(
                    tbl_r.at[vr[j]], rows_v.at[i0 + j, pl.ds(D, D)], sem)
            return ()

        lax.fori_loop(0, BPW // 16, body, ())
        pltpu.make_async_copy(
            out_hbm.at[pl.ds(base, BPW)], rows_v, sem
        ).wait()
        pltpu.sync_copy(rows_v, out_hbm.at[pl.ds(base, BPW)])


TB = 2048


def _mm_body(x01, x23, w_ref, b_ref, o_ref):
    x = jnp.concatenate([x01[...], x23[...]], axis=1)
    acc = lax.dot_general(x, w_ref[...], (((1,), (1,)), ((), ())),
                          preferred_element_type=jnp.float32)
    o_ref[...] = acc + b_ref[...]


_mm = pl.pallas_call(
    _mm_body,
    grid=(B // TB,),
    in_specs=[
        pl.BlockSpec((TB, 2 * D), lambda i: (i, 0)),
        pl.BlockSpec((TB, 2 * D), lambda i: (i, 0)),
        pl.BlockSpec((H, H), lambda i: (0, 0)),
        pl.BlockSpec((1, H), lambda i: (0, 0)),
    ],
    out_specs=pl.BlockSpec((TB, H), lambda i: (i, 0)),
    out_shape=jax.ShapeDtypeStruct((B, H), jnp.float32),
    compiler_params=pltpu.CompilerParams(
        dimension_semantics=("parallel",),
    ),
)


def kernel(emb_genre, emb_artist, emb_album, emb_country, W, b,
           idx_genre, idx_artist, idx_album, idx_country):
    x01, x23 = _sc_gather(
        emb_genre, emb_artist, emb_album, emb_country,
        idx_genre.astype(jnp.int32), idx_artist.astype(jnp.int32),
        idx_album.astype(jnp.int32), idx_country.astype(jnp.int32),
    )
    return _mm(x01, x23, W, b.reshape(1, H))
